# baseline (speedup 1.0000x reference)
import functools

import jax
import jax.numpy as jnp
from jax.experimental import pallas as pl

MASK_RATIO = 0.7
Z_ULPS = 3.0
S_MARGIN = 3e-7


def _rne_bf16_f32(a):
    u = jax.lax.bitcast_convert_type(a, jnp.uint32)
    r = (u + jnp.uint32(0x7FFF) + ((u >> 16) & jnp.uint32(1))) & jnp.uint32(0xFFFF0000)
    return jax.lax.bitcast_convert_type(r, jnp.float32)


def _scores_kernel(x_ref, wt_ref, b_ref, wa_ref, war_ref, ba_ref, s_ref, d_ref):
    z = (
        jnp.dot(x_ref[...], wt_ref[...], preferred_element_type=jnp.float32)
        + b_ref[...]
    )
    h = jnp.tanh(z)
    s_ref[...] = (
        jnp.dot(h, wa_ref[...], preferred_element_type=jnp.float32) + ba_ref[...]
    )
    wob = (Z_ULPS * 1.1920929e-7) * jnp.abs(z) * (1.0 - h * h) + 1e-30
    jump = jnp.abs(_rne_bf16_f32(h + wob) - _rne_bf16_f32(h - wob))
    d_ref[...] = jnp.sum(jnp.abs(war_ref[...]) * jump, axis=1, keepdims=True)


def _mask_softmax_kernel(s_ref, d_ref, w_ref, *, keep_k):
    s = s_ref[...]
    delta = d_ref[...]
    bsz = s.shape[0]
    i = jax.lax.bitcast_convert_type(s, jnp.int32)
    key = jnp.where(i >= 0, i, i ^ jnp.int32(0x7FFFFFFF))

    kk = jnp.int32(keep_k)
    n_nonneg = jnp.sum((key >= 0).astype(jnp.int32), axis=1, keepdims=True)
    sign_base = jnp.where(n_nonneg >= kk, jnp.int32(0), jnp.int32(-(2**31)))

    rv = jnp.zeros((bsz, 1), jnp.int32)
    for bit in range(30, -1, -1):
        t = rv | jnp.int32(1 << bit)
        trial = t | sign_base
        cnt = jnp.sum((key >= trial).astype(jnp.int32), axis=1, keepdims=True)
        rv = jnp.where(cnt >= kk, t, rv)
    kth = rv | sign_base

    n_gt = jnp.sum((key > kth).astype(jnp.int32), axis=1, keepdims=True)
    k_eq = kk - n_gt
    tied = key == kth
    idx = jax.lax.broadcasted_iota(jnp.int32, s.shape, 1)
    cstar = jnp.zeros((bsz, 1), jnp.int32)
    for bit in range(12, -1, -1):
        t2 = cstar | jnp.int32(1 << bit)
        cnt = jnp.sum((tied & (idx >= t2)).astype(jnp.int32), axis=1, keepdims=True)
        cstar = jnp.where(cnt >= k_eq, t2, cstar)

    hold = (key > kth) | (tied & (idx >= cstar))

    masked = ~hold
    intmin = jnp.int32(-(2**31))
    v0 = jnp.max(jnp.where(masked, key, intmin), axis=1, keepdims=True)
    bnd_m_set = masked & (key == v0)
    mrow = jnp.max(jnp.where(bnd_m_set, idx, jnp.int32(-1)), axis=1, keepdims=True)
    is_k = tied & (idx == cstar)
    is_m = bnd_m_set & (idx == mrow)
    pick = lambda cond, v: jnp.sum(jnp.where(cond, v, jnp.float32(0.0)),
                                   axis=1, keepdims=True)
    gap = pick(is_k, s) - pick(is_m, s)
    unc = pick(is_k, delta) + pick(is_m, delta) + jnp.float32(2.0 * S_MARGIN)
    hedge = gap <= unc

    m = jnp.max(s, axis=1, keepdims=True)
    e = jnp.exp(s - m)
    denom = jnp.sum(e, axis=1, keepdims=True)
    w = e / denom
    wo = jnp.where(hold, w, jnp.float32(0.0))
    half = jnp.float32(0.5) * w
    wo = jnp.where(hedge & (is_k | is_m), half, wo)
    w_ref[...] = wo


def _scale_kernel(x_ref, w_ref, o_ref):
    o_ref[...] = x_ref[...] * w_ref[...]


def kernel(x, W, b, wa, ba):
    bsz, t_dim, d = x.shape
    num_mask = int(t_dim * MASK_RATIO)
    keep_k = t_dim - num_mask
    n = bsz * t_dim
    bm = 1024
    grid = n // bm

    xf = x.reshape(n, d)
    wt = W.T
    b2 = b.reshape(1, d)
    wa_col = wa.reshape(d, 1)
    wa_row = wa.reshape(1, d)
    ba2 = ba.reshape(1, 1)

    scores, delta = pl.pallas_call(
        _scores_kernel,
        grid=(grid,),
        in_specs=[
            pl.BlockSpec((bm, d), lambda i: (i, 0)),
            pl.BlockSpec((d, d), lambda i: (0, 0)),
            pl.BlockSpec((1, d), lambda i: (0, 0)),
            pl.BlockSpec((d, 1), lambda i: (0, 0)),
            pl.BlockSpec((1, d), lambda i: (0, 0)),
            pl.BlockSpec((1, 1), lambda i: (0, 0)),
        ],
        out_specs=[
            pl.BlockSpec((bm, 1), lambda i: (i, 0)),
            pl.BlockSpec((bm, 1), lambda i: (i, 0)),
        ],
        out_shape=[
            jax.ShapeDtypeStruct((n, 1), jnp.float32),
            jax.ShapeDtypeStruct((n, 1), jnp.float32),
        ],
    )(xf, wt, b2, wa_col, wa_row, ba2)

    weights = pl.pallas_call(
        functools.partial(_mask_softmax_kernel, keep_k=keep_k),
        in_specs=[
            pl.BlockSpec((bsz, t_dim), lambda: (0, 0)),
            pl.BlockSpec((bsz, t_dim), lambda: (0, 0)),
        ],
        out_specs=pl.BlockSpec((bsz, t_dim), lambda: (0, 0)),
        out_shape=jax.ShapeDtypeStruct((bsz, t_dim), jnp.float32),
    )(scores.reshape(bsz, t_dim), delta.reshape(bsz, t_dim))

    out = pl.pallas_call(
        _scale_kernel,
        grid=(grid,),
        in_specs=[
            pl.BlockSpec((bm, d), lambda i: (i, 0)),
            pl.BlockSpec((bm, 1), lambda i: (i, 0)),
        ],
        out_specs=pl.BlockSpec((bm, d), lambda i: (i, 0)),
        out_shape=jax.ShapeDtypeStruct((n, d), jnp.float32),
    )(xf, weights.reshape(n, 1))

    return (out.reshape(bsz, t_dim, d), weights)


# speedup vs baseline: 1.0163x; 1.0163x over previous
import functools

import jax
import jax.numpy as jnp
from jax.experimental import pallas as pl

MASK_RATIO = 0.7
Z_ULPS = 3.0
S_MARGIN = 3e-7


def _rne_bf16_f32(a):
    u = jax.lax.bitcast_convert_type(a, jnp.uint32)
    r = (u + jnp.uint32(0x7FFF) + ((u >> 16) & jnp.uint32(1))) & jnp.uint32(0xFFFF0000)
    return jax.lax.bitcast_convert_type(r, jnp.float32)


def _scores_kernel(x_ref, wt_ref, b_ref, wa_ref, waa_ref, ba_ref, s_ref, d_ref):
    z = (
        jnp.dot(x_ref[...], wt_ref[...], preferred_element_type=jnp.float32)
        + b_ref[...]
    )
    h = jnp.tanh(z)
    s_ref[...] = (
        jnp.dot(h, wa_ref[...], preferred_element_type=jnp.float32) + ba_ref[...]
    )
    wob = (Z_ULPS * 1.1920929e-7) * jnp.abs(z) * (1.0 - h * h)
    u = jax.lax.bitcast_convert_type(h, jnp.uint32)
    ulp16 = jax.lax.bitcast_convert_type(u & jnp.uint32(0x7F800000),
                                         jnp.float32) * jnp.float32(2.0**-7)
    near = jnp.abs(h - _rne_bf16_f32(h)) >= jnp.float32(0.5) * ulp16 - wob
    jump = jnp.where(near, ulp16, jnp.float32(0.0))
    d_ref[...] = jnp.dot(jump, waa_ref[...],
                         preferred_element_type=jnp.float32) * jnp.float32(1.02)


def _mask_softmax_kernel(s_ref, d_ref, w_ref, *, keep_k):
    s = s_ref[...]
    delta = d_ref[...]
    bsz = s.shape[0]
    i = jax.lax.bitcast_convert_type(s, jnp.int32)
    key = jnp.where(i >= 0, i, i ^ jnp.int32(0x7FFFFFFF))

    kk = jnp.int32(keep_k)
    n_nonneg = jnp.sum((key >= 0).astype(jnp.int32), axis=1, keepdims=True)
    sign_base = jnp.where(n_nonneg >= kk, jnp.int32(0), jnp.int32(-(2**31)))

    rv = jnp.zeros((bsz, 1), jnp.int32)
    for bit in range(30, -1, -1):
        t = rv | jnp.int32(1 << bit)
        trial = t | sign_base
        cnt = jnp.sum((key >= trial).astype(jnp.int32), axis=1, keepdims=True)
        rv = jnp.where(cnt >= kk, t, rv)
    kth = rv | sign_base

    n_gt = jnp.sum((key > kth).astype(jnp.int32), axis=1, keepdims=True)
    k_eq = kk - n_gt
    tied = key == kth
    idx = jax.lax.broadcasted_iota(jnp.int32, s.shape, 1)
    cstar = jnp.zeros((bsz, 1), jnp.int32)
    for bit in range(12, -1, -1):
        t2 = cstar | jnp.int32(1 << bit)
        cnt = jnp.sum((tied & (idx >= t2)).astype(jnp.int32), axis=1, keepdims=True)
        cstar = jnp.where(cnt >= k_eq, t2, cstar)

    hold = (key > kth) | (tied & (idx >= cstar))

    masked = ~hold
    intmin = jnp.int32(-(2**31))
    v0 = jnp.max(jnp.where(masked, key, intmin), axis=1, keepdims=True)
    bnd_m_set = masked & (key == v0)
    mrow = jnp.max(jnp.where(bnd_m_set, idx, jnp.int32(-1)), axis=1, keepdims=True)
    is_k = tied & (idx == cstar)
    is_m = bnd_m_set & (idx == mrow)
    pick = lambda cond, v: jnp.sum(jnp.where(cond, v, jnp.float32(0.0)),
                                   axis=1, keepdims=True)
    gap = pick(is_k, s) - pick(is_m, s)
    unc = pick(is_k, delta) + pick(is_m, delta) + jnp.float32(2.0 * S_MARGIN)
    hedge = gap <= unc

    m = jnp.max(s, axis=1, keepdims=True)
    e = jnp.exp(s - m)
    denom = jnp.sum(e, axis=1, keepdims=True)
    w = e / denom
    wo = jnp.where(hold, w, jnp.float32(0.0))
    half = jnp.float32(0.5) * w
    wo = jnp.where(hedge & (is_k | is_m), half, wo)
    w_ref[...] = wo


def _scale_kernel(x_ref, w_ref, o_ref):
    o_ref[...] = x_ref[...] * w_ref[...]


def kernel(x, W, b, wa, ba):
    bsz, t_dim, d = x.shape
    num_mask = int(t_dim * MASK_RATIO)
    keep_k = t_dim - num_mask
    n = bsz * t_dim
    bm = 1024
    grid = n // bm

    xf = x.reshape(n, d)
    wt = W.T
    b2 = b.reshape(1, d)
    wa_col = wa.reshape(d, 1)
    wa_abs = jnp.abs(wa_col)
    ba2 = ba.reshape(1, 1)

    scores, delta = pl.pallas_call(
        _scores_kernel,
        grid=(grid,),
        in_specs=[
            pl.BlockSpec((bm, d), lambda i: (i, 0)),
            pl.BlockSpec((d, d), lambda i: (0, 0)),
            pl.BlockSpec((1, d), lambda i: (0, 0)),
            pl.BlockSpec((d, 1), lambda i: (0, 0)),
            pl.BlockSpec((d, 1), lambda i: (0, 0)),
            pl.BlockSpec((1, 1), lambda i: (0, 0)),
        ],
        out_specs=[
            pl.BlockSpec((bm, 1), lambda i: (i, 0)),
            pl.BlockSpec((bm, 1), lambda i: (i, 0)),
        ],
        out_shape=[
            jax.ShapeDtypeStruct((n, 1), jnp.float32),
            jax.ShapeDtypeStruct((n, 1), jnp.float32),
        ],
    )(xf, wt, b2, wa_col, wa_abs, ba2)

    weights = pl.pallas_call(
        functools.partial(_mask_softmax_kernel, keep_k=keep_k),
        in_specs=[
            pl.BlockSpec((bsz, t_dim), lambda: (0, 0)),
            pl.BlockSpec((bsz, t_dim), lambda: (0, 0)),
        ],
        out_specs=pl.BlockSpec((bsz, t_dim), lambda: (0, 0)),
        out_shape=jax.ShapeDtypeStruct((bsz, t_dim), jnp.float32),
    )(scores.reshape(bsz, t_dim), delta.reshape(bsz, t_dim))

    out = pl.pallas_call(
        _scale_kernel,
        grid=(grid,),
        in_specs=[
            pl.BlockSpec((bm, d), lambda i: (i, 0)),
            pl.BlockSpec((bm, 1), lambda i: (i, 0)),
        ],
        out_specs=pl.BlockSpec((bm, d), lambda i: (i, 0)),
        out_shape=jax.ShapeDtypeStruct((n, d), jnp.float32),
    )(xf, weights.reshape(n, 1))

    return (out.reshape(bsz, t_dim, d), weights)


# speedup vs baseline: 1.0294x; 1.0129x over previous
import functools

import jax
import jax.numpy as jnp
from jax.experimental import pallas as pl

MASK_RATIO = 0.7
Z_ULPS = 3.0
S_MARGIN = 3e-7


def _rne_bf16_f32(a):
    u = jax.lax.bitcast_convert_type(a, jnp.uint32)
    r = (u + jnp.uint32(0x7FFF) + ((u >> 16) & jnp.uint32(1))) & jnp.uint32(0xFFFF0000)
    return jax.lax.bitcast_convert_type(r, jnp.float32)


def _scores_kernel(x_ref, wt_ref, b_ref, wa_ref, waa_ref, ba_ref, s_ref, d_ref):
    z = (
        jnp.dot(x_ref[...], wt_ref[...], preferred_element_type=jnp.float32)
        + b_ref[...]
    )
    h = jnp.tanh(z)
    s_ref[...] = (
        jnp.dot(h, wa_ref[...], preferred_element_type=jnp.float32) + ba_ref[...]
    )
    u = jax.lax.bitcast_convert_type(h, jnp.int32)
    expo = u & jnp.int32(0x7F800000)
    inv_ulp32 = jax.lax.bitcast_convert_type(jnp.int32(0x4B000000) - expo,
                                             jnp.float32)
    wob_ulps = (jnp.float32(Z_ULPS * 1.1920929e-7) * jnp.abs(z)) * inv_ulp32
    dmid = jnp.abs((u & jnp.int32(0xFFFF)) - jnp.int32(0x8000))
    near = dmid.astype(jnp.float32) <= wob_ulps + jnp.float32(1.0)
    ulp16 = jax.lax.bitcast_convert_type(
        jnp.maximum(expo - jnp.int32(0x03800000), jnp.int32(0)), jnp.float32)
    jump = jnp.where(near, ulp16, jnp.float32(0.0))
    d_ref[...] = jnp.dot(jump, waa_ref[...],
                         preferred_element_type=jnp.float32) * jnp.float32(1.02)


def _mask_softmax_kernel(s_ref, d_ref, w_ref, *, keep_k):
    s = s_ref[...]
    delta = d_ref[...]
    bsz = s.shape[0]
    i = jax.lax.bitcast_convert_type(s, jnp.int32)
    key = jnp.where(i >= 0, i, i ^ jnp.int32(0x7FFFFFFF))

    kk = jnp.int32(keep_k)
    n_nonneg = jnp.sum((key >= 0).astype(jnp.int32), axis=1, keepdims=True)
    sign_base = jnp.where(n_nonneg >= kk, jnp.int32(0), jnp.int32(-(2**31)))

    rv = jnp.zeros((bsz, 1), jnp.int32)
    for bit in range(30, -1, -1):
        t = rv | jnp.int32(1 << bit)
        trial = t | sign_base
        cnt = jnp.sum((key >= trial).astype(jnp.int32), axis=1, keepdims=True)
        rv = jnp.where(cnt >= kk, t, rv)
    kth = rv | sign_base

    n_gt = jnp.sum((key > kth).astype(jnp.int32), axis=1, keepdims=True)
    k_eq = kk - n_gt
    tied = key == kth
    idx = jax.lax.broadcasted_iota(jnp.int32, s.shape, 1)
    cstar = jnp.zeros((bsz, 1), jnp.int32)
    for bit in range(12, -1, -1):
        t2 = cstar | jnp.int32(1 << bit)
        cnt = jnp.sum((tied & (idx >= t2)).astype(jnp.int32), axis=1, keepdims=True)
        cstar = jnp.where(cnt >= k_eq, t2, cstar)

    hold = (key > kth) | (tied & (idx >= cstar))

    masked = ~hold
    intmin = jnp.int32(-(2**31))
    v0 = jnp.max(jnp.where(masked, key, intmin), axis=1, keepdims=True)
    bnd_m_set = masked & (key == v0)
    mrow = jnp.max(jnp.where(bnd_m_set, idx, jnp.int32(-1)), axis=1, keepdims=True)
    is_k = tied & (idx == cstar)
    is_m = bnd_m_set & (idx == mrow)
    pick = lambda cond, v: jnp.sum(jnp.where(cond, v, jnp.float32(0.0)),
                                   axis=1, keepdims=True)
    gap = pick(is_k, s) - pick(is_m, s)
    unc = pick(is_k, delta) + pick(is_m, delta) + jnp.float32(2.0 * S_MARGIN)
    hedge = gap <= unc

    m = jnp.max(s, axis=1, keepdims=True)
    e = jnp.exp(s - m)
    denom = jnp.sum(e, axis=1, keepdims=True)
    w = e / denom
    wo = jnp.where(hold, w, jnp.float32(0.0))
    half = jnp.float32(0.5) * w
    wo = jnp.where(hedge & (is_k | is_m), half, wo)
    w_ref[...] = wo


def _scale_kernel(x_ref, w_ref, o_ref):
    o_ref[...] = x_ref[...] * w_ref[...]


def kernel(x, W, b, wa, ba):
    bsz, t_dim, d = x.shape
    num_mask = int(t_dim * MASK_RATIO)
    keep_k = t_dim - num_mask
    n = bsz * t_dim
    bm = 1024
    grid = n // bm

    xf = x.reshape(n, d)
    wt = W.T
    b2 = b.reshape(1, d)
    wa_col = wa.reshape(d, 1)
    wa_abs = jnp.abs(wa_col)
    ba2 = ba.reshape(1, 1)

    scores, delta = pl.pallas_call(
        _scores_kernel,
        grid=(grid,),
        in_specs=[
            pl.BlockSpec((bm, d), lambda i: (i, 0)),
            pl.BlockSpec((d, d), lambda i: (0, 0)),
            pl.BlockSpec((1, d), lambda i: (0, 0)),
            pl.BlockSpec((d, 1), lambda i: (0, 0)),
            pl.BlockSpec((d, 1), lambda i: (0, 0)),
            pl.BlockSpec((1, 1), lambda i: (0, 0)),
        ],
        out_specs=[
            pl.BlockSpec((bm, 1), lambda i: (i, 0)),
            pl.BlockSpec((bm, 1), lambda i: (i, 0)),
        ],
        out_shape=[
            jax.ShapeDtypeStruct((n, 1), jnp.float32),
            jax.ShapeDtypeStruct((n, 1), jnp.float32),
        ],
    )(xf, wt, b2, wa_col, wa_abs, ba2)

    weights = pl.pallas_call(
        functools.partial(_mask_softmax_kernel, keep_k=keep_k),
        in_specs=[
            pl.BlockSpec((bsz, t_dim), lambda: (0, 0)),
            pl.BlockSpec((bsz, t_dim), lambda: (0, 0)),
        ],
        out_specs=pl.BlockSpec((bsz, t_dim), lambda: (0, 0)),
        out_shape=jax.ShapeDtypeStruct((bsz, t_dim), jnp.float32),
    )(scores.reshape(bsz, t_dim), delta.reshape(bsz, t_dim))

    out = pl.pallas_call(
        _scale_kernel,
        grid=(grid,),
        in_specs=[
            pl.BlockSpec((bm, d), lambda i: (i, 0)),
            pl.BlockSpec((bm, 1), lambda i: (i, 0)),
        ],
        out_specs=pl.BlockSpec((bm, d), lambda i: (i, 0)),
        out_shape=jax.ShapeDtypeStruct((n, d), jnp.float32),
    )(xf, weights.reshape(n, 1))

    return (out.reshape(bsz, t_dim, d), weights)


# speedup vs baseline: 1.1803x; 1.1466x over previous
import functools

import jax
import jax.numpy as jnp
from jax.experimental import pallas as pl
from jax.experimental.pallas import tpu as pltpu

MASK_RATIO = 0.7
Z_ULPS = 3.0
S_MARGIN = 3e-7


def _scores_kernel(x_ref, wt_ref, b_ref, wa_ref, ba_ref, s_ref):
    h = jnp.tanh(
        jnp.dot(x_ref[...], wt_ref[...], preferred_element_type=jnp.float32)
        + b_ref[...]
    )
    s_ref[...] = (
        jnp.dot(h, wa_ref[...], preferred_element_type=jnp.float32) + ba_ref[...]
    )


def _mask_softmax_kernel(s_ref, w_ref, rows_ref, vals_ref, *, keep_k, t_dim):
    s = s_ref[...]
    bsz = s.shape[0]
    i = jax.lax.bitcast_convert_type(s, jnp.int32)
    key = jnp.where(i >= 0, i, i ^ jnp.int32(0x7FFFFFFF))

    kk = jnp.int32(keep_k)
    n_nonneg = jnp.sum((key >= 0).astype(jnp.int32), axis=1, keepdims=True)
    sign_base = jnp.where(n_nonneg >= kk, jnp.int32(0), jnp.int32(-(2**31)))

    rv = jnp.zeros((bsz, 1), jnp.int32)
    for bit in range(30, -1, -1):
        t = rv | jnp.int32(1 << bit)
        trial = t | sign_base
        cnt = jnp.sum((key >= trial).astype(jnp.int32), axis=1, keepdims=True)
        rv = jnp.where(cnt >= kk, t, rv)
    kth = rv | sign_base

    n_gt = jnp.sum((key > kth).astype(jnp.int32), axis=1, keepdims=True)
    k_eq = kk - n_gt
    tied = key == kth
    idx = jax.lax.broadcasted_iota(jnp.int32, s.shape, 1)
    cstar = jnp.zeros((bsz, 1), jnp.int32)
    for bit in range(12, -1, -1):
        t2 = cstar | jnp.int32(1 << bit)
        cnt = jnp.sum((tied & (idx >= t2)).astype(jnp.int32), axis=1, keepdims=True)
        cstar = jnp.where(cnt >= k_eq, t2, cstar)

    hold = (key > kth) | (tied & (idx >= cstar))

    masked = ~hold
    intmin = jnp.int32(-(2**31))
    v0 = jnp.max(jnp.where(masked, key, intmin), axis=1, keepdims=True)
    bnd_m_set = masked & (key == v0)
    mrow = jnp.max(jnp.where(bnd_m_set, idx, jnp.int32(0)), axis=1, keepdims=True)
    is_k = tied & (idx == cstar)
    is_m = bnd_m_set & (idx == mrow)
    pick = lambda cond, v: jnp.sum(jnp.where(cond, v, jnp.float32(0.0)),
                                   axis=1, keepdims=True)
    gap = pick(is_k, s) - pick(is_m, s)

    m = jnp.max(s, axis=1, keepdims=True)
    e = jnp.exp(s - m)
    denom = jnp.sum(e, axis=1, keepdims=True)
    w = e / denom
    w_ref[...] = jnp.where(hold, w, jnp.float32(0.0))

    offs = jax.lax.broadcasted_iota(jnp.int32, (bsz, 1), 0) * jnp.int32(t_dim)
    rows_ref[...] = jnp.concatenate([cstar + offs, mrow + offs], axis=0)
    vals_ref[...] = jnp.concatenate([gap, pick(is_k, w), pick(is_m, w)], axis=0)


def _hedge_kernel(x_hbm, wt_ref, b_ref, waa_ref, rows_ref, vals_ref, w0_ref,
                  w_ref, xrows, sem, *, t_dim):
    bsz = w0_ref.shape[0]
    npair = 2 * bsz
    for r in range(npair):
        pltpu.make_async_copy(
            x_hbm.at[pl.ds(rows_ref[r, 0], 1), :],
            xrows.at[pl.ds(r, 1), :],
            sem,
        ).start()
    for _ in range(npair):
        pltpu.make_async_copy(x_hbm.at[pl.ds(0, 1), :],
                              xrows.at[pl.ds(0, 1), :], sem).wait()

    z = (
        jnp.dot(xrows[...], wt_ref[...], preferred_element_type=jnp.float32)
        + b_ref[...]
    )
    h = jnp.tanh(z)
    u = jax.lax.bitcast_convert_type(h, jnp.int32)
    expo = u & jnp.int32(0x7F800000)
    inv_ulp32 = jax.lax.bitcast_convert_type(jnp.int32(0x4B000000) - expo,
                                             jnp.float32)
    wob_ulps = (jnp.float32(Z_ULPS * 1.1920929e-7) * jnp.abs(z)) * inv_ulp32
    dmid = jnp.abs((u & jnp.int32(0xFFFF)) - jnp.int32(0x8000))
    near = dmid.astype(jnp.float32) <= wob_ulps + jnp.float32(1.0)
    ulp16 = jax.lax.bitcast_convert_type(
        jnp.maximum(expo - jnp.int32(0x03800000), jnp.int32(0)), jnp.float32)
    jump = jnp.where(near, ulp16, jnp.float32(0.0))
    delta = jnp.dot(jump, waa_ref[...],
                    preferred_element_type=jnp.float32) * jnp.float32(1.02)

    d_k = delta[0:bsz, :]
    d_m = delta[bsz:npair, :]
    gap = vals_ref[0:bsz, :]
    w_k = vals_ref[bsz:npair, :]
    w_m = vals_ref[npair:3 * bsz, :]
    hedge = gap <= d_k + d_m + jnp.float32(2.0 * S_MARGIN)

    riota = jax.lax.broadcasted_iota(jnp.int32, (bsz, 1), 0)
    klocal = jnp.zeros((bsz, 1), jnp.int32)
    mlocal = jnp.zeros((bsz, 1), jnp.int32)
    for bb in range(bsz):
        klocal = jnp.where(riota == bb, rows_ref[bb, 0] - bb * t_dim, klocal)
        mlocal = jnp.where(riota == bb, rows_ref[bsz + bb, 0] - bb * t_dim,
                           mlocal)
    idx = jax.lax.broadcasted_iota(jnp.int32, w0_ref.shape, 1)
    w = w0_ref[...]
    half = jnp.float32(0.5)
    w = jnp.where(hedge & (idx == klocal), half * w_k, w)
    w = jnp.where(hedge & (idx == mlocal), half * w_m, w)
    w_ref[...] = w


def _scale_kernel(x_ref, w_ref, o_ref):
    o_ref[...] = x_ref[...] * w_ref[...]


def kernel(x, W, b, wa, ba):
    bsz, t_dim, d = x.shape
    num_mask = int(t_dim * MASK_RATIO)
    keep_k = t_dim - num_mask
    n = bsz * t_dim
    bm = 1024
    grid = n // bm

    xf = x.reshape(n, d)
    wt = W.T
    b2 = b.reshape(1, d)
    wa_col = wa.reshape(d, 1)
    wa_abs = jnp.abs(wa_col)
    ba2 = ba.reshape(1, 1)

    scores = pl.pallas_call(
        _scores_kernel,
        grid=(grid,),
        in_specs=[
            pl.BlockSpec((bm, d), lambda i: (i, 0)),
            pl.BlockSpec((d, d), lambda i: (0, 0)),
            pl.BlockSpec((1, d), lambda i: (0, 0)),
            pl.BlockSpec((d, 1), lambda i: (0, 0)),
            pl.BlockSpec((1, 1), lambda i: (0, 0)),
        ],
        out_specs=pl.BlockSpec((bm, 1), lambda i: (i, 0)),
        out_shape=jax.ShapeDtypeStruct((n, 1), jnp.float32),
    )(xf, wt, b2, wa_col, ba2)

    w0, prows, pvals = pl.pallas_call(
        functools.partial(_mask_softmax_kernel, keep_k=keep_k, t_dim=t_dim),
        in_specs=[pl.BlockSpec((bsz, t_dim), lambda: (0, 0))],
        out_specs=[
            pl.BlockSpec((bsz, t_dim), lambda: (0, 0)),
            pl.BlockSpec((2 * bsz, 1), lambda: (0, 0)),
            pl.BlockSpec((3 * bsz, 1), lambda: (0, 0)),
        ],
        out_shape=[
            jax.ShapeDtypeStruct((bsz, t_dim), jnp.float32),
            jax.ShapeDtypeStruct((2 * bsz, 1), jnp.int32),
            jax.ShapeDtypeStruct((3 * bsz, 1), jnp.float32),
        ],
    )(scores.reshape(bsz, t_dim))

    weights = pl.pallas_call(
        functools.partial(_hedge_kernel, t_dim=t_dim),
        in_specs=[
            pl.BlockSpec(memory_space=pl.ANY),
            pl.BlockSpec((d, d), lambda: (0, 0)),
            pl.BlockSpec((1, d), lambda: (0, 0)),
            pl.BlockSpec((d, 1), lambda: (0, 0)),
            pl.BlockSpec(memory_space=pltpu.SMEM),
            pl.BlockSpec((3 * bsz, 1), lambda: (0, 0)),
            pl.BlockSpec((bsz, t_dim), lambda: (0, 0)),
        ],
        out_specs=pl.BlockSpec((bsz, t_dim), lambda: (0, 0)),
        out_shape=jax.ShapeDtypeStruct((bsz, t_dim), jnp.float32),
        scratch_shapes=[
            pltpu.VMEM((2 * bsz, d), jnp.float32),
            pltpu.SemaphoreType.DMA,
        ],
    )(xf, wt, b2, wa_abs, prows, pvals, w0)

    out = pl.pallas_call(
        _scale_kernel,
        grid=(grid,),
        in_specs=[
            pl.BlockSpec((bm, d), lambda i: (i, 0)),
            pl.BlockSpec((bm, 1), lambda i: (i, 0)),
        ],
        out_specs=pl.BlockSpec((bm, d), lambda i: (i, 0)),
        out_shape=jax.ShapeDtypeStruct((n, d), jnp.float32),
    )(xf, weights.reshape(n, 1))

    return (out.reshape(bsz, t_dim, d), weights)
